# initial kernel scaffold (unmeasured)
import jax
import jax.numpy as jnp
from jax import lax
from jax.experimental import pallas as pl
from jax.experimental.pallas import tpu as pltpu

N_DEV = 32
M = 2048
N = 2048
M_PER = M // N_DEV
N_HOPS = N_DEV - 1


def kernel(A, B):
    a16 = A.astype(jnp.bfloat16)
    b16 = B.astype(jnp.bfloat16)

    def body(a_ref, b_ref, out_ref, p_ref, comm_ref, send_ref,
             send_sems, recv_sems):
        my = lax.axis_index("i")
        left = lax.rem(my + N_DEV - 1, N_DEV)
        right = lax.rem(my + 1, N_DEV)

        barrier_sem = pltpu.get_barrier_semaphore()
        for nbr in (left, right):
            pl.semaphore_signal(
                barrier_sem, inc=1,
                device_id=(nbr,), device_id_type=pl.DeviceIdType.MESH,
            )
        pl.semaphore_wait(barrier_sem, 2)

        p_ref[...] = jnp.dot(
            a_ref[...], b_ref[...], preferred_element_type=jnp.float32
        )

        for s in range(N_HOPS):
            d = lax.rem(my + N_DEV - s - 1, N_DEV)
            chunk = p_ref[pl.ds(d * M_PER, M_PER), :]
            if s == 0:
                send_ref[...] = chunk
            else:
                send_ref[...] = comm_ref[s - 1] + chunk
            rdma = pltpu.make_async_remote_copy(
                src_ref=send_ref,
                dst_ref=comm_ref.at[s],
                send_sem=send_sems.at[s],
                recv_sem=recv_sems.at[s],
                device_id=(right,),
                device_id_type=pl.DeviceIdType.MESH,
            )
            rdma.start()
            rdma.wait()

        out_ref[...] = (
            comm_ref[N_HOPS - 1] + p_ref[pl.ds(my * M_PER, M_PER), :]
        )

    return pl.pallas_call(
        body,
        out_shape=jax.ShapeDtypeStruct((M_PER, N), jnp.float32),
        in_specs=[
            pl.BlockSpec(memory_space=pltpu.VMEM),
            pl.BlockSpec(memory_space=pltpu.VMEM),
        ],
        out_specs=pl.BlockSpec(memory_space=pltpu.VMEM),
        scratch_shapes=[
            pltpu.VMEM((M, N), jnp.float32),
            pltpu.VMEM((N_HOPS, M_PER, N), jnp.float32),
            pltpu.VMEM((M_PER, N), jnp.float32),
            pltpu.SemaphoreType.DMA((N_HOPS,)),
            pltpu.SemaphoreType.DMA((N_HOPS,)),
        ],
        compiler_params=pltpu.CompilerParams(collective_id=0),
    )(a16, b16)


# baseline (device time: 280453 ns/iter reference)
import jax
import jax.numpy as jnp
from jax import lax
from jax.experimental import pallas as pl
from jax.experimental.pallas import tpu as pltpu

N_DEV = 32
M = 2048
N = 2048
M_PER = M // N_DEV
N_HOPS = N_DEV - 1


def kernel(A, B):
    a16 = A.astype(jnp.bfloat16)
    b16 = B.astype(jnp.bfloat16)

    def body(a_ref, b_ref, out_ref, comm_ref, send_ref,
             send_sems, recv_sems):
        my = lax.axis_index("i")
        left = lax.rem(my + N_DEV - 1, N_DEV)
        right = lax.rem(my + 1, N_DEV)

        barrier_sem = pltpu.get_barrier_semaphore()
        for nbr in (left, right):
            pl.semaphore_signal(
                barrier_sem, inc=1,
                device_id=(nbr,), device_id_type=pl.DeviceIdType.MESH,
            )
        pl.semaphore_wait(barrier_sem, 2)

        def pchunk(d):
            return jnp.dot(
                a_ref[pl.ds(d * M_PER, M_PER), :], b_ref[...],
                preferred_element_type=jnp.float32,
            )

        for s in range(N_HOPS):
            d = lax.rem(my + N_DEV - s - 1, N_DEV)
            chunk = pchunk(d)
            if s == 0:
                send_ref[...] = chunk
            else:
                send_ref[...] = comm_ref[s - 1] + chunk
            rdma = pltpu.make_async_remote_copy(
                src_ref=send_ref,
                dst_ref=comm_ref.at[s],
                send_sem=send_sems.at[s],
                recv_sem=recv_sems.at[s],
                device_id=(right,),
                device_id_type=pl.DeviceIdType.MESH,
            )
            rdma.start()
            rdma.wait()

        out_ref[...] = comm_ref[N_HOPS - 1] + pchunk(my)

    return pl.pallas_call(
        body,
        out_shape=jax.ShapeDtypeStruct((M_PER, N), jnp.float32),
        in_specs=[
            pl.BlockSpec(memory_space=pltpu.VMEM),
            pl.BlockSpec(memory_space=pltpu.VMEM),
        ],
        out_specs=pl.BlockSpec(memory_space=pltpu.VMEM),
        scratch_shapes=[
            pltpu.VMEM((N_HOPS, M_PER, N), jnp.float32),
            pltpu.VMEM((M_PER, N), jnp.float32),
            pltpu.SemaphoreType.DMA((N_HOPS,)),
            pltpu.SemaphoreType.DMA((N_HOPS,)),
        ],
        compiler_params=pltpu.CompilerParams(
            collective_id=0, vmem_limit_bytes=100 * 1024 * 1024,
        ),
    )(a16, b16)


# device time: 151209 ns/iter; 1.8547x vs baseline; 1.8547x over previous
import jax
import jax.numpy as jnp
from jax import lax
from jax.experimental import pallas as pl
from jax.experimental.pallas import tpu as pltpu

N_DEV = 32
M = 2048
N = 2048
M_PER = M // N_DEV
N_R = 16
N_L = 15


def kernel(A, B):
    a16 = A.astype(jnp.bfloat16)
    b16 = B.astype(jnp.bfloat16)

    def body(a_ref, b_ref, out_ref, commr_ref, comml_ref,
             sendr_ref, sendl_ref,
             sendr_sems, recvr_sems, sendl_sems, recvl_sems):
        my = lax.axis_index("i")
        left = lax.rem(my + N_DEV - 1, N_DEV)
        right = lax.rem(my + 1, N_DEV)

        barrier_sem = pltpu.get_barrier_semaphore()
        for nbr in (left, right):
            pl.semaphore_signal(
                barrier_sem, inc=1,
                device_id=(nbr,), device_id_type=pl.DeviceIdType.MESH,
            )
        pl.semaphore_wait(barrier_sem, 2)

        def pchunk(d):
            return jnp.dot(
                a_ref[pl.ds(d * M_PER, M_PER), :], b_ref[...],
                preferred_element_type=jnp.float32,
            )

        for s in range(N_R):
            d_r = lax.rem(my + N_R - s, N_DEV)
            cr = pchunk(d_r)
            if s == 0:
                sendr_ref[...] = cr.astype(jnp.bfloat16)
            else:
                sendr_ref[...] = (
                    commr_ref[s - 1].astype(jnp.float32) + cr
                ).astype(jnp.bfloat16)
            rdma_r = pltpu.make_async_remote_copy(
                src_ref=sendr_ref,
                dst_ref=commr_ref.at[s],
                send_sem=sendr_sems.at[s],
                recv_sem=recvr_sems.at[s],
                device_id=(right,),
                device_id_type=pl.DeviceIdType.MESH,
            )
            rdma_r.start()

            if s < N_L:
                d_l = lax.rem(my + N_DEV - N_L + s, N_DEV)
                cl = pchunk(d_l)
                if s == 0:
                    sendl_ref[...] = cl.astype(jnp.bfloat16)
                else:
                    sendl_ref[...] = (
                        comml_ref[s - 1].astype(jnp.float32) + cl
                    ).astype(jnp.bfloat16)
                rdma_l = pltpu.make_async_remote_copy(
                    src_ref=sendl_ref,
                    dst_ref=comml_ref.at[s],
                    send_sem=sendl_sems.at[s],
                    recv_sem=recvl_sems.at[s],
                    device_id=(left,),
                    device_id_type=pl.DeviceIdType.MESH,
                )
                rdma_l.start()
                rdma_l.wait()
            rdma_r.wait()

        out_ref[...] = (
            pchunk(my)
            + commr_ref[N_R - 1].astype(jnp.float32)
            + comml_ref[N_L - 1].astype(jnp.float32)
        )

    return pl.pallas_call(
        body,
        out_shape=jax.ShapeDtypeStruct((M_PER, N), jnp.float32),
        in_specs=[
            pl.BlockSpec(memory_space=pltpu.VMEM),
            pl.BlockSpec(memory_space=pltpu.VMEM),
        ],
        out_specs=pl.BlockSpec(memory_space=pltpu.VMEM),
        scratch_shapes=[
            pltpu.VMEM((N_R, M_PER, N), jnp.bfloat16),
            pltpu.VMEM((N_L, M_PER, N), jnp.bfloat16),
            pltpu.VMEM((M_PER, N), jnp.bfloat16),
            pltpu.VMEM((M_PER, N), jnp.bfloat16),
            pltpu.SemaphoreType.DMA((N_R,)),
            pltpu.SemaphoreType.DMA((N_R,)),
            pltpu.SemaphoreType.DMA((N_L,)),
            pltpu.SemaphoreType.DMA((N_L,)),
        ],
        compiler_params=pltpu.CompilerParams(
            collective_id=0, vmem_limit_bytes=100 * 1024 * 1024,
        ),
    )(a16, b16)


# device time: 120665 ns/iter; 2.3242x vs baseline; 1.2531x over previous
import jax
import jax.numpy as jnp
from jax import lax
from jax.experimental import pallas as pl
from jax.experimental.pallas import tpu as pltpu

N_DEV = 32
M = 2048
N = 2048
M_PER = M // N_DEV
N_R = 16
N_L = 15


def kernel(A, B):
    a16 = A.astype(jnp.bfloat16)
    b16 = B.astype(jnp.bfloat16)

    def body(a_ref, b_ref, out_ref, commr_ref, comml_ref,
             sendr_ref, sendl_ref,
             sendr_sems, recvr_sems, sendl_sems, recvl_sems):
        my = lax.axis_index("i")
        left = lax.rem(my + N_DEV - 1, N_DEV)
        right = lax.rem(my + 1, N_DEV)

        barrier_sem = pltpu.get_barrier_semaphore()
        for nbr in (left, right):
            pl.semaphore_signal(
                barrier_sem, inc=1,
                device_id=(nbr,), device_id_type=pl.DeviceIdType.MESH,
            )
        pl.semaphore_wait(barrier_sem, 2)

        def pchunk(d):
            return jnp.dot(
                a_ref[pl.ds(d * M_PER, M_PER), :], b_ref[...],
                preferred_element_type=jnp.float32,
            )

        descs_r = []
        descs_l = []
        for s in range(N_R):
            cr = pchunk(lax.rem(my + N_R - s, N_DEV))
            do_l = s < N_L
            if do_l:
                cl = pchunk(lax.rem(my + N_DEV - N_L + s, N_DEV))

            if s > 0:
                descs_r[s - 1].wait_recv()
                cr = cr + commr_ref[s - 1].astype(jnp.float32)
            sendr_ref[s, :, :] = cr.astype(jnp.bfloat16)
            rdma_r = pltpu.make_async_remote_copy(
                src_ref=sendr_ref.at[s],
                dst_ref=commr_ref.at[s],
                send_sem=sendr_sems.at[s],
                recv_sem=recvr_sems.at[s],
                device_id=(right,),
                device_id_type=pl.DeviceIdType.MESH,
            )
            rdma_r.start()
            descs_r.append(rdma_r)

            if do_l:
                if s > 0:
                    descs_l[s - 1].wait_recv()
                    cl = cl + comml_ref[s - 1].astype(jnp.float32)
                sendl_ref[s, :, :] = cl.astype(jnp.bfloat16)
                rdma_l = pltpu.make_async_remote_copy(
                    src_ref=sendl_ref.at[s],
                    dst_ref=comml_ref.at[s],
                    send_sem=sendl_sems.at[s],
                    recv_sem=recvl_sems.at[s],
                    device_id=(left,),
                    device_id_type=pl.DeviceIdType.MESH,
                )
                rdma_l.start()
                descs_l.append(rdma_l)

        c_my = pchunk(my)
        descs_r[N_R - 1].wait_recv()
        descs_l[N_L - 1].wait_recv()
        out_ref[...] = (
            c_my
            + commr_ref[N_R - 1].astype(jnp.float32)
            + comml_ref[N_L - 1].astype(jnp.float32)
        )

        for d in descs_r:
            d.wait_send()
        for d in descs_l:
            d.wait_send()

    return pl.pallas_call(
        body,
        out_shape=jax.ShapeDtypeStruct((M_PER, N), jnp.float32),
        in_specs=[
            pl.BlockSpec(memory_space=pltpu.VMEM),
            pl.BlockSpec(memory_space=pltpu.VMEM),
        ],
        out_specs=pl.BlockSpec(memory_space=pltpu.VMEM),
        scratch_shapes=[
            pltpu.VMEM((N_R, M_PER, N), jnp.bfloat16),
            pltpu.VMEM((N_L, M_PER, N), jnp.bfloat16),
            pltpu.VMEM((N_R, M_PER, N), jnp.bfloat16),
            pltpu.VMEM((N_L, M_PER, N), jnp.bfloat16),
            pltpu.SemaphoreType.DMA((N_R,)),
            pltpu.SemaphoreType.DMA((N_R,)),
            pltpu.SemaphoreType.DMA((N_L,)),
            pltpu.SemaphoreType.DMA((N_L,)),
        ],
        compiler_params=pltpu.CompilerParams(
            collective_id=0, vmem_limit_bytes=100 * 1024 * 1024,
        ),
    )(a16, b16)


# device time: 111526 ns/iter; 2.5147x vs baseline; 1.0819x over previous
import jax
import jax.numpy as jnp
import numpy as np
from jax import lax
from jax.experimental import pallas as pl
from jax.experimental.pallas import tpu as pltpu

N_DEV = 32
M = 2048
N = 2048
M_PER = M // N_DEV
N_R = 16
N_L = 15


def _ring_tables():
    coords = [(x, y, z) for z in range(4) for y in range(4) for x in range(2)]
    logical = []
    for z in range(4):
        for yi, y in enumerate(range(4)):
            row = [(x, y, z) for x in range(2)]
            if yi % 2:
                row.reverse()
            logical.extend(row)
    cycle = []
    for z in range(4):
        ys = range(4) if z % 2 == 0 else range(3, -1, -1)
        cycle.extend((0, y, z) for y in ys)
    for z in range(3, -1, -1):
        ys = range(4) if z % 2 else range(3, -1, -1)
        cycle.extend((1, y, z) for y in ys)
    coord_to_log = {c: i for i, c in enumerate(logical)}
    pi = np.array([coord_to_log[c] for c in cycle], dtype=np.int32)
    t = np.empty(N_DEV, dtype=np.int32)
    t[pi] = np.arange(N_DEV, dtype=np.int32)
    return pi, t


_PI, _T = _ring_tables()


def kernel(A, B):
    a16 = A.astype(jnp.bfloat16)
    b16 = B.astype(jnp.bfloat16)

    my_log = lax.axis_index("i")
    pi = jnp.asarray(_PI)
    my_t = jnp.asarray(_T)[my_log]
    right_log = pi[(my_t + 1) % N_DEV]
    left_log = pi[(my_t + N_DEV - 1) % N_DEV]
    meta = jnp.stack([my_t, right_log, left_log]).astype(jnp.int32)

    def body(a_ref, b_ref, pi_ref, meta_ref, out_ref, commr_ref, comml_ref,
             sendr_ref, sendl_ref,
             sendr_sems, recvr_sems, sendl_sems, recvl_sems):
        my = lax.axis_index("i")
        my_t = meta_ref[0]
        right = meta_ref[1]
        left = meta_ref[2]

        barrier_sem = pltpu.get_barrier_semaphore()
        for nbr in (left, right):
            pl.semaphore_signal(
                barrier_sem, inc=1,
                device_id=(nbr,), device_id_type=pl.DeviceIdType.MESH,
            )
        pl.semaphore_wait(barrier_sem, 2)

        def pchunk(d):
            return jnp.dot(
                a_ref[pl.ds(d * M_PER, M_PER), :], b_ref[...],
                preferred_element_type=jnp.float32,
            )

        descs_r = []
        descs_l = []
        for s in range(N_R):
            cr = pchunk(pi_ref[lax.rem(my_t + N_R - s, N_DEV)])
            do_l = s < N_L
            if do_l:
                cl = pchunk(pi_ref[lax.rem(my_t + N_DEV - N_L + s, N_DEV)])

            if s > 0:
                descs_r[s - 1].wait_recv()
                cr = cr + commr_ref[s - 1].astype(jnp.float32)
            sendr_ref[s, :, :] = cr.astype(jnp.bfloat16)
            rdma_r = pltpu.make_async_remote_copy(
                src_ref=sendr_ref.at[s],
                dst_ref=commr_ref.at[s],
                send_sem=sendr_sems.at[s],
                recv_sem=recvr_sems.at[s],
                device_id=(right,),
                device_id_type=pl.DeviceIdType.MESH,
            )
            rdma_r.start()
            descs_r.append(rdma_r)

            if do_l:
                if s > 0:
                    descs_l[s - 1].wait_recv()
                    cl = cl + comml_ref[s - 1].astype(jnp.float32)
                sendl_ref[s, :, :] = cl.astype(jnp.bfloat16)
                rdma_l = pltpu.make_async_remote_copy(
                    src_ref=sendl_ref.at[s],
                    dst_ref=comml_ref.at[s],
                    send_sem=sendl_sems.at[s],
                    recv_sem=recvl_sems.at[s],
                    device_id=(left,),
                    device_id_type=pl.DeviceIdType.MESH,
                )
                rdma_l.start()
                descs_l.append(rdma_l)

        c_my = pchunk(my)
        descs_r[N_R - 1].wait_recv()
        descs_l[N_L - 1].wait_recv()
        out_ref[...] = (
            c_my
            + commr_ref[N_R - 1].astype(jnp.float32)
            + comml_ref[N_L - 1].astype(jnp.float32)
        )

        for d in descs_r:
            d.wait_send()
        for d in descs_l:
            d.wait_send()

    return pl.pallas_call(
        body,
        out_shape=jax.ShapeDtypeStruct((M_PER, N), jnp.float32),
        in_specs=[
            pl.BlockSpec(memory_space=pltpu.VMEM),
            pl.BlockSpec(memory_space=pltpu.VMEM),
            pl.BlockSpec(memory_space=pltpu.SMEM),
            pl.BlockSpec(memory_space=pltpu.SMEM),
        ],
        out_specs=pl.BlockSpec(memory_space=pltpu.VMEM),
        scratch_shapes=[
            pltpu.VMEM((N_R, M_PER, N), jnp.bfloat16),
            pltpu.VMEM((N_L, M_PER, N), jnp.bfloat16),
            pltpu.VMEM((N_R, M_PER, N), jnp.bfloat16),
            pltpu.VMEM((N_L, M_PER, N), jnp.bfloat16),
            pltpu.SemaphoreType.DMA((N_R,)),
            pltpu.SemaphoreType.DMA((N_R,)),
            pltpu.SemaphoreType.DMA((N_L,)),
            pltpu.SemaphoreType.DMA((N_L,)),
        ],
        compiler_params=pltpu.CompilerParams(
            collective_id=0, vmem_limit_bytes=100 * 1024 * 1024,
        ),
    )(a16, b16, pi, meta)


# device time: 86059 ns/iter; 3.2588x vs baseline; 1.2959x over previous
import jax
import jax.numpy as jnp
import numpy as np
from jax import lax
from jax.experimental import pallas as pl
from jax.experimental.pallas import tpu as pltpu

N_DEV = 32
M = 2048
N = 2048
K = 1024
M_PER = M // N_DEV
N_R = 16
N_L = 15
B_SUB = 2
NB = N // B_SUB
G = 4
MG = M // G


def _ring_tables():
    coords = [(x, y, z) for z in range(4) for y in range(4) for x in range(2)]
    logical = []
    for z in range(4):
        for yi, y in enumerate(range(4)):
            row = [(x, y, z) for x in range(2)]
            if yi % 2:
                row.reverse()
            logical.extend(row)
    cycle = []
    for z in range(4):
        ys = range(4) if z % 2 == 0 else range(3, -1, -1)
        cycle.extend((0, y, z) for y in ys)
    for z in range(3, -1, -1):
        ys = range(4) if z % 2 else range(3, -1, -1)
        cycle.extend((1, y, z) for y in ys)
    coord_to_log = {c: i for i, c in enumerate(logical)}
    pi = np.array([coord_to_log[c] for c in cycle], dtype=np.int32)
    t = np.empty(N_DEV, dtype=np.int32)
    t[pi] = np.arange(N_DEV, dtype=np.int32)
    return pi, t


_PI, _T = _ring_tables()


def kernel(A, B):
    a16 = A.astype(jnp.bfloat16)
    b16 = B.astype(jnp.bfloat16)

    my_log = lax.axis_index("i")
    pi = jnp.asarray(_PI)
    my_t = jnp.asarray(_T)[my_log]
    right_log = pi[(my_t + 1) % N_DEV]
    left_log = pi[(my_t + N_DEV - 1) % N_DEV]
    meta = jnp.stack([my_t, right_log, left_log]).astype(jnp.int32)

    def body(a_ref, b_ref, pi_ref, meta_ref, out_ref, p_ref,
             commr_ref, comml_ref,
             sendr_sems, recvr_sems, sendl_sems, recvl_sems):
        my = lax.axis_index("i")
        my_t = meta_ref[0]
        right = meta_ref[1]
        left = meta_ref[2]

        barrier_sem = pltpu.get_barrier_semaphore()
        for nbr in (left, right):
            pl.semaphore_signal(
                barrier_sem, inc=1,
                device_id=(nbr,), device_id_type=pl.DeviceIdType.MESH,
            )
        pl.semaphore_wait(barrier_sem, 2)

        for g in range(G):
            p_ref[pl.ds(g * MG, MG), :] = jnp.dot(
                a_ref[pl.ds(g * MG, MG), :], b_ref[...],
                preferred_element_type=jnp.float32,
            ).astype(jnp.bfloat16)

        def rows(d):
            return pl.ds(d * M_PER, M_PER)

        def cols(b):
            return pl.ds(b * NB, NB)

        descs_r = [[None] * B_SUB for _ in range(N_R)]
        descs_l = [[None] * B_SUB for _ in range(N_L)]
        for s in range(N_R):
            d_r = pi_ref[lax.rem(my_t + N_R - s, N_DEV)]
            do_l = s < N_L
            if do_l:
                d_l = pi_ref[lax.rem(my_t + N_DEV - N_L + s, N_DEV)]
            for b in range(B_SUB):
                for is_left in (False, True):
                    if is_left and not do_l:
                        continue
                    d = d_l if is_left else d_r
                    comm = comml_ref if is_left else commr_ref
                    ssems = sendl_sems if is_left else sendr_sems
                    rsems = recvl_sems if is_left else recvr_sems
                    descs = descs_l if is_left else descs_r
                    nbr = left if is_left else right
                    if s == 0:
                        src = p_ref.at[rows(d), cols(b)]
                    else:
                        descs[s - 1][b].wait_recv()
                        acc = (
                            p_ref[rows(d), cols(b)].astype(jnp.float32)
                            + comm[s - 1, :, cols(b)].astype(jnp.float32)
                        )
                        comm[s - 1, :, cols(b)] = acc.astype(jnp.bfloat16)
                        src = comm.at[s - 1, :, cols(b)]
                    rdma = pltpu.make_async_remote_copy(
                        src_ref=src,
                        dst_ref=comm.at[s, :, cols(b)],
                        send_sem=ssems.at[s * B_SUB + b],
                        recv_sem=rsems.at[s * B_SUB + b],
                        device_id=(nbr,),
                        device_id_type=pl.DeviceIdType.MESH,
                    )
                    rdma.start()
                    descs[s][b] = rdma

        for b in range(B_SUB):
            descs_r[N_R - 1][b].wait_recv()
            descs_l[N_L - 1][b].wait_recv()
            out_ref[:, cols(b)] = (
                p_ref[rows(my), cols(b)].astype(jnp.float32)
                + commr_ref[N_R - 1, :, cols(b)].astype(jnp.float32)
                + comml_ref[N_L - 1, :, cols(b)].astype(jnp.float32)
            )

        for hop in descs_r:
            for d in hop:
                d.wait_send()
        for hop in descs_l:
            for d in hop:
                d.wait_send()

    return pl.pallas_call(
        body,
        out_shape=jax.ShapeDtypeStruct((M_PER, N), jnp.float32),
        in_specs=[
            pl.BlockSpec(memory_space=pltpu.VMEM),
            pl.BlockSpec(memory_space=pltpu.VMEM),
            pl.BlockSpec(memory_space=pltpu.SMEM),
            pl.BlockSpec(memory_space=pltpu.SMEM),
        ],
        out_specs=pl.BlockSpec(memory_space=pltpu.VMEM),
        scratch_shapes=[
            pltpu.VMEM((M, N), jnp.bfloat16),
            pltpu.VMEM((N_R, M_PER, N), jnp.bfloat16),
            pltpu.VMEM((N_L, M_PER, N), jnp.bfloat16),
            pltpu.SemaphoreType.DMA((N_R * B_SUB,)),
            pltpu.SemaphoreType.DMA((N_R * B_SUB,)),
            pltpu.SemaphoreType.DMA((N_L * B_SUB,)),
            pltpu.SemaphoreType.DMA((N_L * B_SUB,)),
        ],
        compiler_params=pltpu.CompilerParams(
            collective_id=0, vmem_limit_bytes=100 * 1024 * 1024,
        ),
    )(a16, b16, pi, meta)


# device time: 85949 ns/iter; 3.2630x vs baseline; 1.0013x over previous
import jax
import jax.numpy as jnp
import numpy as np
from jax import lax
from jax.experimental import pallas as pl
from jax.experimental.pallas import tpu as pltpu

N_DEV = 32
M = 2048
N = 2048
K = 1024
M_PER = M // N_DEV
N_R = 16
N_L = 15
B_SUB = 2
NB = N // B_SUB
G = 4
MG = M // G


def _ring_tables():
    coords = [(x, y, z) for z in range(4) for y in range(4) for x in range(2)]
    logical = []
    for z in range(4):
        for yi, y in enumerate(range(4)):
            row = [(x, y, z) for x in range(2)]
            if yi % 2:
                row.reverse()
            logical.extend(row)
    cycle = []
    for z in range(4):
        ys = range(4) if z % 2 == 0 else range(3, -1, -1)
        cycle.extend((0, y, z) for y in ys)
    for z in range(3, -1, -1):
        ys = range(4) if z % 2 else range(3, -1, -1)
        cycle.extend((1, y, z) for y in ys)
    coord_to_log = {c: i for i, c in enumerate(logical)}
    pi = np.array([coord_to_log[c] for c in cycle], dtype=np.int32)
    t = np.empty(N_DEV, dtype=np.int32)
    t[pi] = np.arange(N_DEV, dtype=np.int32)
    return pi, t


_PI, _T = _ring_tables()


def kernel(A, B):
    a16 = A.astype(jnp.bfloat16)
    b16 = B.astype(jnp.bfloat16)

    my_log = lax.axis_index("i")
    pi = jnp.asarray(_PI)
    my_t = jnp.asarray(_T)[my_log]
    right_log = pi[(my_t + 1) % N_DEV]
    left_log = pi[(my_t + N_DEV - 1) % N_DEV]
    meta = jnp.stack([my_t, right_log, left_log]).astype(jnp.int32)

    def body(a_ref, b_ref, pi_ref, meta_ref, out_ref, p_ref,
             commr_ref, comml_ref,
             sendr_sems, recvr_sems, sendl_sems, recvl_sems):
        my = lax.axis_index("i")
        my_t = meta_ref[0]
        right = meta_ref[1]
        left = meta_ref[2]

        barrier_sem = pltpu.get_barrier_semaphore()
        for nbr in (left, right):
            pl.semaphore_signal(
                barrier_sem, inc=1,
                device_id=(nbr,), device_id_type=pl.DeviceIdType.MESH,
            )
        pl.semaphore_wait(barrier_sem, 2)

        def rows(d):
            return pl.ds(d * M_PER, M_PER)

        def cols(b):
            return pl.ds(b * NB, NB)

        for g in range(G):
            p_ref[pl.ds(g * MG, MG), :] = jnp.dot(
                a_ref[pl.ds(g * MG, MG), :], b_ref[...],
                preferred_element_type=jnp.float32,
            ).astype(jnp.bfloat16)

        descs_r = [[None] * B_SUB for _ in range(N_R)]
        descs_l = [[None] * B_SUB for _ in range(N_L)]
        for s in range(N_R):
            d_r = pi_ref[lax.rem(my_t + N_R - s, N_DEV)]
            do_l = s < N_L
            if do_l:
                d_l = pi_ref[lax.rem(my_t + N_DEV - N_L + s, N_DEV)]
            for b in range(B_SUB):
                for is_left in (False, True):
                    if is_left and not do_l:
                        continue
                    d = d_l if is_left else d_r
                    comm = comml_ref if is_left else commr_ref
                    ssems = sendl_sems if is_left else sendr_sems
                    rsems = recvl_sems if is_left else recvr_sems
                    descs = descs_l if is_left else descs_r
                    nbr = left if is_left else right
                    if s == 0:
                        src = p_ref.at[rows(d), cols(b)]
                    else:
                        descs[s - 1][b].wait_recv()
                        acc = (
                            p_ref[rows(d), cols(b)].astype(jnp.float32)
                            + comm[s - 1, :, cols(b)].astype(jnp.float32)
                        )
                        comm[s - 1, :, cols(b)] = acc.astype(jnp.bfloat16)
                        src = comm.at[s - 1, :, cols(b)]
                    rdma = pltpu.make_async_remote_copy(
                        src_ref=src,
                        dst_ref=comm.at[s, :, cols(b)],
                        send_sem=ssems.at[s * B_SUB + b],
                        recv_sem=rsems.at[s * B_SUB + b],
                        device_id=(nbr,),
                        device_id_type=pl.DeviceIdType.MESH,
                    )
                    rdma.start()
                    descs[s][b] = rdma

        for b in range(B_SUB):
            descs_r[N_R - 1][b].wait_recv()
            descs_l[N_L - 1][b].wait_recv()
            out_ref[:, cols(b)] = (
                p_ref[rows(my), cols(b)].astype(jnp.float32)
                + commr_ref[N_R - 1, :, cols(b)].astype(jnp.float32)
                + comml_ref[N_L - 1, :, cols(b)].astype(jnp.float32)
            )

        for hop in descs_r:
            for d in hop:
                d.wait_send()
        for hop in descs_l:
            for d in hop:
                d.wait_send()

    return pl.pallas_call(
        body,
        out_shape=jax.ShapeDtypeStruct((M_PER, N), jnp.float32),
        in_specs=[
            pl.BlockSpec(memory_space=pltpu.VMEM),
            pl.BlockSpec(memory_space=pltpu.VMEM),
            pl.BlockSpec(memory_space=pltpu.SMEM),
            pl.BlockSpec(memory_space=pltpu.SMEM),
        ],
        out_specs=pl.BlockSpec(memory_space=pltpu.VMEM),
        scratch_shapes=[
            pltpu.VMEM((M, N), jnp.bfloat16),
            pltpu.VMEM((N_R, M_PER, N), jnp.bfloat16),
            pltpu.VMEM((N_L, M_PER, N), jnp.bfloat16),
            pltpu.SemaphoreType.DMA((N_R * B_SUB,)),
            pltpu.SemaphoreType.DMA((N_R * B_SUB,)),
            pltpu.SemaphoreType.DMA((N_L * B_SUB,)),
            pltpu.SemaphoreType.DMA((N_L * B_SUB,)),
        ],
        compiler_params=pltpu.CompilerParams(
            collective_id=0, vmem_limit_bytes=100 * 1024 * 1024,
        ),
    )(a16, b16, pi, meta)
